# initial kernel scaffold (unmeasured)
import jax
import jax.numpy as jnp
from jax import lax
from jax.experimental import pallas as pl
from jax.experimental.pallas import tpu as pltpu

N_DEV = 4


def _gelu(y):
    c = 0.7978845608028654
    return 0.5 * y * (1.0 + jnp.tanh(c * (y + 0.044715 * y * y * y)))


def kernel(x, w_mat):
    m_per, k = x.shape
    _, n_per = w_mat.shape

    def body(x_ref, w_ref, out_ref, comm_ref, send_sems, recv_sems):
        my = lax.axis_index("i")
        left = (my - 1) % N_DEV
        right = (my + 1) % N_DEV

        barrier_sem = pltpu.get_barrier_semaphore()
        for nbr in (left, right):
            pl.semaphore_signal(
                barrier_sem, inc=1,
                device_id=(nbr,), device_id_type=pl.DeviceIdType.MESH,
            )
        pl.semaphore_wait(barrier_sem, 2)

        rdmas = []
        rdma0 = pltpu.make_async_remote_copy(
            src_ref=x_ref,
            dst_ref=comm_ref.at[0],
            send_sem=send_sems.at[0],
            recv_sem=recv_sems.at[0],
            device_id=(right,),
            device_id_type=pl.DeviceIdType.MESH,
        )
        rdma0.start()
        rdmas.append(rdma0)

        w = w_ref[...]
        out_ref[pl.ds(my * m_per, m_per), :] = _gelu(
            jnp.dot(x_ref[...], w, preferred_element_type=jnp.float32)
        )

        for h in range(1, N_DEV - 1):
            rdmas[h - 1].wait_recv()
            rdma = pltpu.make_async_remote_copy(
                src_ref=comm_ref.at[h - 1],
                dst_ref=comm_ref.at[h],
                send_sem=send_sems.at[h],
                recv_sem=recv_sems.at[h],
                device_id=(right,),
                device_id_type=pl.DeviceIdType.MESH,
            )
            rdma.start()
            rdmas.append(rdma)
            origin = (my - h) % N_DEV
            out_ref[pl.ds(origin * m_per, m_per), :] = _gelu(
                jnp.dot(comm_ref[h - 1], w, preferred_element_type=jnp.float32)
            )

        rdmas[N_DEV - 2].wait_recv()
        origin = (my - (N_DEV - 1)) % N_DEV
        out_ref[pl.ds(origin * m_per, m_per), :] = _gelu(
            jnp.dot(comm_ref[N_DEV - 2], w, preferred_element_type=jnp.float32)
        )

        for r in rdmas:
            r.wait_send()

    out_shape = jax.ShapeDtypeStruct((N_DEV * m_per, n_per), jnp.float32)
    return pl.pallas_call(
        body,
        out_shape=out_shape,
        in_specs=[
            pl.BlockSpec(memory_space=pltpu.VMEM),
            pl.BlockSpec(memory_space=pltpu.VMEM),
        ],
        out_specs=pl.BlockSpec(memory_space=pltpu.VMEM),
        scratch_shapes=[
            pltpu.VMEM((N_DEV - 1, m_per, k), jnp.float32),
            pltpu.SemaphoreType.DMA((N_DEV - 1,)),
            pltpu.SemaphoreType.DMA((N_DEV - 1,)),
        ],
        compiler_params=pltpu.CompilerParams(collective_id=0),
    )(x, w_mat)


# baseline (device time: 577386 ns/iter reference)
import jax
import jax.numpy as jnp
from jax import lax
from jax.experimental import pallas as pl
from jax.experimental.pallas import tpu as pltpu

N_DEV = 4


def _gelu(y):
    c = 0.7978845608028654
    return 0.5 * y * (1.0 + jnp.tanh(c * (y + 0.044715 * y * y * y)))


def kernel(x, w_mat):
    m_per, k = x.shape
    _, n_per = w_mat.shape
    half = m_per // 2

    def body(x_ref, w_ref, out_ref, hbm_ref,
             stream_ref, send_sems, recv_sems, copy_sems):
        my = lax.axis_index("i")
        left = (my - 1) % N_DEV
        right = (my + 1) % N_DEV

        barrier_sem = pltpu.get_barrier_semaphore()
        for nbr in (left, right):
            pl.semaphore_signal(
                barrier_sem, inc=1,
                device_id=(nbr,), device_id_type=pl.DeviceIdType.MESH,
            )
        pl.semaphore_wait(barrier_sem, 2)

        def remote_send(src, h):
            rdma = pltpu.make_async_remote_copy(
                src_ref=src,
                dst_ref=hbm_ref.at[h],
                send_sem=send_sems.at[h],
                recv_sem=recv_sems.at[h],
                device_id=(right,),
                device_id_type=pl.DeviceIdType.MESH,
            )
            rdma.start()
            return rdma

        w = w_ref[...]

        def gemm_rows(block, row_start):
            out_ref[pl.ds(row_start, half), :] = _gelu(
                jnp.dot(block, w, preferred_element_type=jnp.float32)
            )

        rdmas = [remote_send(x_ref, 0)]
        for t in range(2):
            gemm_rows(x_ref[pl.ds(t * half, half), :], my * m_per + t * half)

        for h in range(N_DEV - 1):
            rdmas[h].wait_recv()
            if h + 1 < N_DEV - 1:
                rdmas.append(remote_send(hbm_ref.at[h], h + 1))
            origin = (my - 1 - h) % N_DEV
            cps = []
            for t in range(2):
                cp = pltpu.make_async_copy(
                    hbm_ref.at[h, pl.ds(t * half, half), :],
                    stream_ref.at[t],
                    copy_sems.at[t],
                )
                cp.start()
                cps.append(cp)
            for t in range(2):
                cps[t].wait()
                gemm_rows(stream_ref[t], origin * m_per + t * half)

        for r in rdmas:
            r.wait_send()

    out_shapes = [
        jax.ShapeDtypeStruct((N_DEV * m_per, n_per), jnp.float32),
        jax.ShapeDtypeStruct((N_DEV - 1, m_per, k), jnp.float32),
    ]
    out, _ = pl.pallas_call(
        body,
        out_shape=out_shapes,
        in_specs=[
            pl.BlockSpec(memory_space=pltpu.MemorySpace.VMEM),
            pl.BlockSpec(memory_space=pltpu.MemorySpace.VMEM),
        ],
        out_specs=[
            pl.BlockSpec(memory_space=pltpu.MemorySpace.VMEM),
            pl.BlockSpec(memory_space=pltpu.MemorySpace.HBM),
        ],
        scratch_shapes=[
            pltpu.VMEM((2, half, k), jnp.float32),
            pltpu.SemaphoreType.DMA((N_DEV - 1,)),
            pltpu.SemaphoreType.DMA((N_DEV - 1,)),
            pltpu.SemaphoreType.DMA((2,)),
        ],
        compiler_params=pltpu.CompilerParams(
            collective_id=0,
            vmem_limit_bytes=63 * 1024 * 1024,
        ),
    )(x, w_mat)
    return out


# device time: 310548 ns/iter; 1.8592x vs baseline; 1.8592x over previous
import jax
import jax.numpy as jnp
from jax import lax
from jax.experimental import pallas as pl
from jax.experimental.pallas import tpu as pltpu

N_DEV = 4


def _gelu(y):
    c = 0.7978845608028654
    return 0.5 * y * (1.0 + jnp.tanh(c * (y + 0.044715 * y * y * y)))


def kernel(x, w_mat):
    m_per, k = x.shape
    _, n_per = w_mat.shape
    half = m_per // 2

    SLOT_L, SLOT_R, SLOT_O = 0, 1, 2

    def body(x_ref, w_ref, out_ref, hbm_ref,
             stream_ref, send_sems, recv_sems, copy_sems):
        my = lax.axis_index("i")
        left = (my - 1) % N_DEV
        right = (my + 1) % N_DEV

        barrier_sem = pltpu.get_barrier_semaphore()
        for nbr in (left, right):
            pl.semaphore_signal(
                barrier_sem, inc=1,
                device_id=(nbr,), device_id_type=pl.DeviceIdType.MESH,
            )
        pl.semaphore_wait(barrier_sem, 2)

        def remote_send(src, dst, sem_idx, target):
            rdma = pltpu.make_async_remote_copy(
                src_ref=src,
                dst_ref=dst,
                send_sem=send_sems.at[sem_idx],
                recv_sem=recv_sems.at[sem_idx],
                device_id=(target,),
                device_id_type=pl.DeviceIdType.MESH,
            )
            rdma.start()
            return rdma

        w = w_ref[...]

        def gemm_rows(block, row_start):
            out_ref[pl.ds(row_start, half), :] = _gelu(
                jnp.dot(block, w, preferred_element_type=jnp.float32)
            )

        def compute_hbm_chunk(slot, origin):
            cps = []
            for t in range(2):
                cp = pltpu.make_async_copy(
                    hbm_ref.at[slot, pl.ds(t * half, half), :],
                    stream_ref.at[t],
                    copy_sems.at[t],
                )
                cp.start()
                cps.append(cp)
            for t in range(2):
                cps[t].wait()
                gemm_rows(stream_ref[t], origin * m_per + t * half)

        p1_to_right = remote_send(x_ref, hbm_ref.at[SLOT_L], 0, right)
        p1_to_left = remote_send(x_ref, hbm_ref.at[SLOT_R], 1, left)

        for t in range(2):
            gemm_rows(x_ref[pl.ds(t * half, half), :], my * m_per + t * half)

        p1_to_right.wait_recv()
        p2_to_right = remote_send(
            hbm_ref.at[SLOT_L, pl.ds(0, half), :],
            hbm_ref.at[SLOT_O, pl.ds(0, half), :],
            2, right,
        )
        compute_hbm_chunk(SLOT_L, (my - 1) % N_DEV)

        p1_to_left.wait_recv()
        p2_to_left = remote_send(
            hbm_ref.at[SLOT_R, pl.ds(half, half), :],
            hbm_ref.at[SLOT_O, pl.ds(half, half), :],
            3, left,
        )
        compute_hbm_chunk(SLOT_R, (my + 1) % N_DEV)

        opp = (my + 2) % N_DEV
        cp0 = pltpu.make_async_copy(
            hbm_ref.at[SLOT_O, pl.ds(0, half), :], stream_ref.at[0],
            copy_sems.at[0],
        )
        p2_to_right.wait_recv()
        cp0.start()
        cp1 = pltpu.make_async_copy(
            hbm_ref.at[SLOT_O, pl.ds(half, half), :], stream_ref.at[1],
            copy_sems.at[1],
        )
        p2_to_left.wait_recv()
        cp1.start()
        cp0.wait()
        gemm_rows(stream_ref[0], opp * m_per)
        cp1.wait()
        gemm_rows(stream_ref[1], opp * m_per + half)

        for r in (p1_to_right, p1_to_left, p2_to_right, p2_to_left):
            r.wait_send()

    out_shapes = [
        jax.ShapeDtypeStruct((N_DEV * m_per, n_per), jnp.float32),
        jax.ShapeDtypeStruct((N_DEV - 1, m_per, k), jnp.float32),
    ]
    out, _ = pl.pallas_call(
        body,
        out_shape=out_shapes,
        in_specs=[
            pl.BlockSpec(memory_space=pltpu.MemorySpace.VMEM),
            pl.BlockSpec(memory_space=pltpu.MemorySpace.VMEM),
        ],
        out_specs=[
            pl.BlockSpec(memory_space=pltpu.MemorySpace.VMEM),
            pl.BlockSpec(memory_space=pltpu.MemorySpace.HBM),
        ],
        scratch_shapes=[
            pltpu.VMEM((2, half, k), jnp.float32),
            pltpu.SemaphoreType.DMA((4,)),
            pltpu.SemaphoreType.DMA((4,)),
            pltpu.SemaphoreType.DMA((2,)),
        ],
        compiler_params=pltpu.CompilerParams(
            collective_id=0,
            vmem_limit_bytes=63 * 1024 * 1024,
        ),
    )(x, w_mat)
    return out


# device time: 307106 ns/iter; 1.8801x vs baseline; 1.0112x over previous
import jax
import jax.numpy as jnp
from jax import lax
from jax.experimental import pallas as pl
from jax.experimental.pallas import tpu as pltpu

N_DEV = 4


def _gelu(y):
    c = 0.7978845608028654
    return 0.5 * y * (1.0 + jnp.tanh(c * (y + 0.044715 * y * y * y)))


def kernel(x, w_mat):
    m_per, k = x.shape
    _, n_per = w_mat.shape
    half = m_per // 2
    quarter = m_per // 4

    SLOT_L, SLOT_R, SLOT_O = 0, 1, 2

    def body(x_ref, w_ref, out_ref, hbm_ref,
             stream_ref, send_sems, recv_sems, copy_sems):
        my = lax.axis_index("i")
        left = (my - 1) % N_DEV
        right = (my + 1) % N_DEV

        barrier_sem = pltpu.get_barrier_semaphore()
        for nbr in (left, right):
            pl.semaphore_signal(
                barrier_sem, inc=1,
                device_id=(nbr,), device_id_type=pl.DeviceIdType.MESH,
            )
        pl.semaphore_wait(barrier_sem, 2)

        def remote_send(src, dst, sem_idx, target):
            rdma = pltpu.make_async_remote_copy(
                src_ref=src,
                dst_ref=dst,
                send_sem=send_sems.at[sem_idx],
                recv_sem=recv_sems.at[sem_idx],
                device_id=(target,),
                device_id_type=pl.DeviceIdType.MESH,
            )
            rdma.start()
            return rdma

        w = w_ref[...]

        def gemm_half(block, row_start):
            out_ref[pl.ds(row_start, half), :] = _gelu(
                jnp.dot(block, w, preferred_element_type=jnp.float32)
            )

        def gemm_quarter(block, row_start):
            out_ref[pl.ds(row_start, quarter), :] = _gelu(
                jnp.dot(block, w, preferred_element_type=jnp.float32)
            )

        p1_to_right = remote_send(x_ref, hbm_ref.at[SLOT_L], 0, right)
        p1_to_left = remote_send(x_ref, hbm_ref.at[SLOT_R], 1, left)

        for t in range(2):
            gemm_half(x_ref[pl.ds(t * half, half), :], my * m_per + t * half)

        descs = {}

        def pre_L():
            p1_to_right.wait_recv()
            for q, sem in ((0, 2), (1, 3)):
                descs[sem] = remote_send(
                    hbm_ref.at[SLOT_L, pl.ds(q * quarter, quarter), :],
                    hbm_ref.at[SLOT_O, pl.ds(q * quarter, quarter), :],
                    sem, right,
                )

        def pre_R():
            p1_to_left.wait_recv()
            for q, sem in ((2, 4), (3, 5)):
                descs[sem] = remote_send(
                    hbm_ref.at[SLOT_R, pl.ds(q * quarter, quarter), :],
                    hbm_ref.at[SLOT_O, pl.ds(q * quarter, quarter), :],
                    sem, left,
                )

        org_L = (my - 1) % N_DEV
        org_R = (my + 1) % N_DEV
        org_O = (my + 2) % N_DEV
        jobs = (
            [(pre_L if q == 0 else None, SLOT_L, q * quarter,
              org_L * m_per + q * quarter) for q in range(4)]
            + [(pre_R if q == 0 else None, SLOT_R, q * quarter,
                org_R * m_per + q * quarter) for q in range(4)]
            + [
                (lambda: descs[2].wait_recv(), SLOT_O, 0 * quarter,
                 org_O * m_per + 0 * quarter),
                (lambda: descs[4].wait_recv(), SLOT_O, 2 * quarter,
                 org_O * m_per + 2 * quarter),
                (lambda: descs[3].wait_recv(), SLOT_O, 1 * quarter,
                 org_O * m_per + 1 * quarter),
                (lambda: descs[5].wait_recv(), SLOT_O, 3 * quarter,
                 org_O * m_per + 3 * quarter),
            ]
        )

        cps = []
        for i, (pre, slot, srow, orow) in enumerate(jobs):
            if pre is not None:
                pre()
            cp = pltpu.make_async_copy(
                hbm_ref.at[slot, pl.ds(srow, quarter), :],
                stream_ref.at[i % 2],
                copy_sems.at[i % 2],
            )
            cp.start()
            cps.append(cp)
            if i >= 1:
                cps[i - 1].wait()
                gemm_quarter(stream_ref[(i - 1) % 2], jobs[i - 1][3])
        cps[-1].wait()
        gemm_quarter(stream_ref[(len(jobs) - 1) % 2], jobs[-1][3])

        for r in (p1_to_right, p1_to_left,
                  descs[2], descs[3], descs[4], descs[5]):
            r.wait_send()

    out_shapes = [
        jax.ShapeDtypeStruct((N_DEV * m_per, n_per), jnp.float32),
        jax.ShapeDtypeStruct((N_DEV - 1, m_per, k), jnp.float32),
    ]
    out, _ = pl.pallas_call(
        body,
        out_shape=out_shapes,
        in_specs=[
            pl.BlockSpec(memory_space=pltpu.MemorySpace.VMEM),
            pl.BlockSpec(memory_space=pltpu.MemorySpace.VMEM),
        ],
        out_specs=[
            pl.BlockSpec(memory_space=pltpu.MemorySpace.VMEM),
            pl.BlockSpec(memory_space=pltpu.MemorySpace.HBM),
        ],
        scratch_shapes=[
            pltpu.VMEM((2, quarter, k), jnp.float32),
            pltpu.SemaphoreType.DMA((6,)),
            pltpu.SemaphoreType.DMA((6,)),
            pltpu.SemaphoreType.DMA((2,)),
        ],
        compiler_params=pltpu.CompilerParams(
            collective_id=0,
            vmem_limit_bytes=63 * 1024 * 1024,
        ),
    )(x, w_mat)
    return out


# device time: 222387 ns/iter; 2.5963x vs baseline; 1.3810x over previous
import jax
import jax.numpy as jnp
from jax import lax
from jax.experimental import pallas as pl
from jax.experimental.pallas import tpu as pltpu

N_DEV = 4


def _gelu(y):
    c = 0.7978845608028654
    return 0.5 * y * (1.0 + jnp.tanh(c * (y + 0.044715 * y * y * y)))


def kernel(x, w_mat):
    m_per, k = x.shape
    _, n_per = w_mat.shape
    half = m_per // 2
    n_half = n_per // 2

    SLOT_L, SLOT_R, SLOT_O = 0, 1, 2
    P1_R, P1_L, P2_R, P2_L, PC_L, PC_R, PC_O = range(7)

    def body(x_ref, w_ref, out_ref, hbm_ref,
             wbuf_ref, stage_ref, send_sems, recv_sems, copy_sems):
        my = lax.axis_index("i")
        left = (my - 1) % N_DEV
        right = (my + 1) % N_DEV
        opp = (my + 2) % N_DEV

        barrier_sem = pltpu.get_barrier_semaphore()
        for nbr in (left, right, opp):
            pl.semaphore_signal(
                barrier_sem, inc=1,
                device_id=(nbr,), device_id_type=pl.DeviceIdType.MESH,
            )
        pl.semaphore_wait(barrier_sem, 3)

        def remote_send(src, dst, sem_idx, target):
            rdma = pltpu.make_async_remote_copy(
                src_ref=src,
                dst_ref=dst,
                send_sem=send_sems.at[sem_idx],
                recv_sem=recv_sems.at[sem_idx],
                device_id=(target,),
                device_id_type=pl.DeviceIdType.MESH,
            )
            rdma.start()
            return rdma

        p1_r = remote_send(w_ref, hbm_ref.at[SLOT_L], P1_R, right)
        p1_l = remote_send(w_ref, hbm_ref.at[SLOT_R], P1_L, left)

        for t in range(2):
            out_ref[pl.ds(my * m_per + t * half, half), :] = _gelu(
                jnp.dot(x_ref[pl.ds(t * half, half), :], w_ref[...],
                        preferred_element_type=jnp.float32)
            )

        def piece_for(slot, stage_idx, sem_idx, target):
            cps = []
            for h in range(2):
                cp = pltpu.make_async_copy(
                    hbm_ref.at[slot, :, pl.ds(h * n_half, n_half)],
                    wbuf_ref.at[h],
                    copy_sems.at[h],
                )
                cp.start()
                cps.append(cp)
            for h in range(2):
                cps[h].wait()
                for t in range(2):
                    stage_ref[stage_idx, pl.ds(t * half, half),
                              pl.ds(h * n_half, n_half)] = _gelu(
                        jnp.dot(x_ref[pl.ds(t * half, half), :], wbuf_ref[h],
                                preferred_element_type=jnp.float32)
                    )
            return remote_send(
                stage_ref.at[stage_idx],
                out_ref.at[pl.ds(my * m_per, m_per), :],
                sem_idx, target,
            )

        p1_r.wait_recv()
        p2_r = remote_send(
            hbm_ref.at[SLOT_L, :, pl.ds(0, n_half)],
            hbm_ref.at[SLOT_O, :, pl.ds(0, n_half)],
            P2_R, right,
        )
        pc_l = piece_for(SLOT_L, 0, PC_L, left)

        p1_l.wait_recv()
        p2_l = remote_send(
            hbm_ref.at[SLOT_R, :, pl.ds(n_half, n_half)],
            hbm_ref.at[SLOT_O, :, pl.ds(n_half, n_half)],
            P2_L, left,
        )
        pc_r = piece_for(SLOT_R, 1, PC_R, right)

        p2_r.wait_recv()
        p2_l.wait_recv()
        pc_o = piece_for(SLOT_O, 2, PC_O, opp)

        pc_l.wait_recv()
        pc_r.wait_recv()
        pc_o.wait_recv()
        for r in (p1_r, p1_l, p2_r, p2_l, pc_l, pc_r, pc_o):
            r.wait_send()

    out_shapes = [
        jax.ShapeDtypeStruct((N_DEV * m_per, n_per), jnp.float32),
        jax.ShapeDtypeStruct((3, k, n_per), jnp.float32),
    ]
    out, _ = pl.pallas_call(
        body,
        out_shape=out_shapes,
        in_specs=[
            pl.BlockSpec(memory_space=pltpu.MemorySpace.VMEM),
            pl.BlockSpec(memory_space=pltpu.MemorySpace.VMEM),
        ],
        out_specs=[
            pl.BlockSpec(memory_space=pltpu.MemorySpace.VMEM),
            pl.BlockSpec(memory_space=pltpu.MemorySpace.HBM),
        ],
        scratch_shapes=[
            pltpu.VMEM((2, k, n_half), jnp.float32),
            pltpu.VMEM((3, m_per, n_per), jnp.float32),
            pltpu.SemaphoreType.DMA((7,)),
            pltpu.SemaphoreType.DMA((7,)),
            pltpu.SemaphoreType.DMA((2,)),
        ],
        compiler_params=pltpu.CompilerParams(
            collective_id=0,
            vmem_limit_bytes=63 * 1024 * 1024,
        ),
    )(x, w_mat)
    return out


# device time: 208812 ns/iter; 2.7651x vs baseline; 1.0650x over previous
import jax
import jax.numpy as jnp
from jax import lax
from jax.experimental import pallas as pl
from jax.experimental.pallas import tpu as pltpu

N_DEV = 4


def _gelu(y):
    c = 0.7978845608028654
    return 0.5 * y * (1.0 + jnp.tanh(c * (y + 0.044715 * y * y * y)))


def kernel(x, w_mat):
    m_per, k = x.shape
    _, n_per = w_mat.shape
    half = m_per // 2
    n_half = n_per // 2

    SLOT_L, SLOT_R = 0, 1
    (P1_R0, P1_R1, P1_L0, P1_L1,
     P2_R, P2_L, PC_L, PC_R, PC_O) = range(9)

    def body(x_ref, w_ref, out_ref, hbm_ref,
             opp_ref, wbuf_ref, stage_ref, send_sems, recv_sems, copy_sems):
        my = lax.axis_index("i")
        left = (my - 1) % N_DEV
        right = (my + 1) % N_DEV
        opp = (my + 2) % N_DEV

        barrier_sem = pltpu.get_barrier_semaphore()
        for nbr in (left, right, opp):
            pl.semaphore_signal(
                barrier_sem, inc=1,
                device_id=(nbr,), device_id_type=pl.DeviceIdType.MESH,
            )
        pl.semaphore_wait(barrier_sem, 3)

        def remote_send(src, dst, sem_idx, target):
            rdma = pltpu.make_async_remote_copy(
                src_ref=src,
                dst_ref=dst,
                send_sem=send_sems.at[sem_idx],
                recv_sem=recv_sems.at[sem_idx],
                device_id=(target,),
                device_id_type=pl.DeviceIdType.MESH,
            )
            rdma.start()
            return rdma

        def cols(hv):
            return pl.ds(hv * n_half, n_half)

        p1_r0 = remote_send(w_ref.at[:, cols(0)],
                            hbm_ref.at[SLOT_L, :, cols(0)], P1_R0, right)
        p1_l0 = remote_send(w_ref.at[:, cols(0)],
                            hbm_ref.at[SLOT_R, :, cols(0)], P1_L0, left)
        p1_r1 = remote_send(w_ref.at[:, cols(1)],
                            hbm_ref.at[SLOT_L, :, cols(1)], P1_R1, right)
        p1_l1 = remote_send(w_ref.at[:, cols(1)],
                            hbm_ref.at[SLOT_R, :, cols(1)], P1_L1, left)

        for t in range(2):
            out_ref[pl.ds(my * m_per + t * half, half), :] = _gelu(
                jnp.dot(x_ref[pl.ds(t * half, half), :], w_ref[...],
                        preferred_element_type=jnp.float32)
            )

        def dots(w_block, stage_idx, hv):
            for t in range(2):
                stage_ref[stage_idx, pl.ds(t * half, half), cols(hv)] = _gelu(
                    jnp.dot(x_ref[pl.ds(t * half, half), :], w_block,
                            preferred_element_type=jnp.float32)
                )

        def start_wcopy(slot, hv, wslot):
            cp = pltpu.make_async_copy(
                hbm_ref.at[slot, :, cols(hv)],
                wbuf_ref.at[wslot],
                copy_sems.at[wslot],
            )
            cp.start()
            return cp

        def send_piece(stage_idx, sem_idx, target):
            return remote_send(
                stage_ref.at[stage_idx],
                out_ref.at[pl.ds(my * m_per, m_per), :],
                sem_idx, target,
            )

        p1_r0.wait_recv()
        p2_r = remote_send(hbm_ref.at[SLOT_L, :, cols(0)],
                           opp_ref.at[:, cols(0)], P2_R, right)
        cp_l = start_wcopy(SLOT_L, 0, 0)
        p1_l0.wait_recv()
        cp_r = start_wcopy(SLOT_R, 0, 1)
        cp_l.wait()
        dots(wbuf_ref[0], 0, 0)
        p1_r1.wait_recv()
        cp_l = start_wcopy(SLOT_L, 1, 0)
        cp_r.wait()
        dots(wbuf_ref[1], 1, 0)
        p1_l1.wait_recv()
        p2_l = remote_send(hbm_ref.at[SLOT_R, :, cols(1)],
                           opp_ref.at[:, cols(1)], P2_L, left)
        cp_r = start_wcopy(SLOT_R, 1, 1)
        cp_l.wait()
        dots(wbuf_ref[0], 0, 1)
        pc_l = send_piece(0, PC_L, left)
        cp_r.wait()
        dots(wbuf_ref[1], 1, 1)
        pc_r = send_piece(1, PC_R, right)

        p2_r.wait_recv()
        dots(opp_ref[:, cols(0)], 2, 0)
        p2_l.wait_recv()
        dots(opp_ref[:, cols(1)], 2, 1)
        pc_o = send_piece(2, PC_O, opp)

        pc_l.wait_recv()
        pc_r.wait_recv()
        pc_o.wait_recv()
        for r in (p1_r0, p1_r1, p1_l0, p1_l1, p2_r, p2_l, pc_l, pc_r, pc_o):
            r.wait_send()

    out_shapes = [
        jax.ShapeDtypeStruct((N_DEV * m_per, n_per), jnp.float32),
        jax.ShapeDtypeStruct((2, k, n_per), jnp.float32),
    ]
    out, _ = pl.pallas_call(
        body,
        out_shape=out_shapes,
        in_specs=[
            pl.BlockSpec(memory_space=pltpu.MemorySpace.VMEM),
            pl.BlockSpec(memory_space=pltpu.MemorySpace.VMEM),
        ],
        out_specs=[
            pl.BlockSpec(memory_space=pltpu.MemorySpace.VMEM),
            pl.BlockSpec(memory_space=pltpu.MemorySpace.HBM),
        ],
        scratch_shapes=[
            pltpu.VMEM((k, n_per), jnp.float32),
            pltpu.VMEM((2, k, n_half), jnp.float32),
            pltpu.VMEM((3, m_per, n_per), jnp.float32),
            pltpu.SemaphoreType.DMA((9,)),
            pltpu.SemaphoreType.DMA((9,)),
            pltpu.SemaphoreType.DMA((2,)),
        ],
        compiler_params=pltpu.CompilerParams(
            collective_id=0,
            vmem_limit_bytes=63 * 1024 * 1024,
        ),
    )(x, w_mat)
    return out


# device time: 119399 ns/iter; 4.8358x vs baseline; 1.7489x over previous
import jax
import jax.numpy as jnp
from jax import lax
from jax.experimental import pallas as pl
from jax.experimental.pallas import tpu as pltpu

N_DEV = 4


def _gelu(y):
    c = 0.7978845608028654
    return 0.5 * y * (1.0 + jnp.tanh(c * (y + 0.044715 * y * y * y)))


def kernel(x, w_mat):
    m_per, k = x.shape
    _, n_per = w_mat.shape
    half = m_per // 2
    n_half = n_per // 2
    bf16 = jnp.bfloat16

    SLOT_L, SLOT_R = 0, 1
    PIECE_FROM_R, PIECE_FROM_L, PIECE_FROM_O = 0, 1, 2
    (P1_R0, P1_R1, P1_L0, P1_L1,
     P2_R, P2_L, PC_L, PC_R, PC_O) = range(9)

    def body(x_ref, w_ref, out_ref, hbm_ref,
             wsrc_ref, opp_ref, wbuf_ref, stage_ref, pieces_ref,
             send_sems, recv_sems, copy_sems):
        my = lax.axis_index("i")
        left = (my - 1) % N_DEV
        right = (my + 1) % N_DEV
        opp = (my + 2) % N_DEV

        wsrc_ref[...] = w_ref[...].astype(bf16)

        barrier_sem = pltpu.get_barrier_semaphore()
        for nbr in (left, right, opp):
            pl.semaphore_signal(
                barrier_sem, inc=1,
                device_id=(nbr,), device_id_type=pl.DeviceIdType.MESH,
            )
        pl.semaphore_wait(barrier_sem, 3)

        def remote_send(src, dst, sem_idx, target):
            rdma = pltpu.make_async_remote_copy(
                src_ref=src,
                dst_ref=dst,
                send_sem=send_sems.at[sem_idx],
                recv_sem=recv_sems.at[sem_idx],
                device_id=(target,),
                device_id_type=pl.DeviceIdType.MESH,
            )
            rdma.start()
            return rdma

        def cols(hv):
            return pl.ds(hv * n_half, n_half)

        p1_r0 = remote_send(wsrc_ref.at[:, cols(0)],
                            hbm_ref.at[SLOT_L, :, cols(0)], P1_R0, right)
        p1_l0 = remote_send(wsrc_ref.at[:, cols(0)],
                            hbm_ref.at[SLOT_R, :, cols(0)], P1_L0, left)
        p1_r1 = remote_send(wsrc_ref.at[:, cols(1)],
                            hbm_ref.at[SLOT_L, :, cols(1)], P1_R1, right)
        p1_l1 = remote_send(wsrc_ref.at[:, cols(1)],
                            hbm_ref.at[SLOT_R, :, cols(1)], P1_L1, left)

        for t in range(2):
            out_ref[pl.ds(my * m_per + t * half, half), :] = _gelu(
                jnp.dot(x_ref[pl.ds(t * half, half), :], w_ref[...],
                        preferred_element_type=jnp.float32)
            )

        def dots(w_block, stage_idx, hv):
            for t in range(2):
                stage_ref[stage_idx, pl.ds(t * half, half), cols(hv)] = _gelu(
                    jnp.dot(x_ref[pl.ds(t * half, half), :],
                            w_block.astype(jnp.float32),
                            preferred_element_type=jnp.float32)
                ).astype(bf16)

        def start_wcopy(slot, hv, wslot):
            cp = pltpu.make_async_copy(
                hbm_ref.at[slot, :, cols(hv)],
                wbuf_ref.at[wslot],
                copy_sems.at[wslot],
            )
            cp.start()
            return cp

        def send_piece(stage_idx, dst_slot, sem_idx, target):
            return remote_send(
                stage_ref.at[stage_idx],
                pieces_ref.at[dst_slot],
                sem_idx, target,
            )

        p1_r0.wait_recv()
        p2_r = remote_send(hbm_ref.at[SLOT_L, :, cols(0)],
                           opp_ref.at[:, cols(0)], P2_R, right)
        cp_l = start_wcopy(SLOT_L, 0, 0)
        p1_l0.wait_recv()
        cp_r = start_wcopy(SLOT_R, 0, 1)
        cp_l.wait()
        dots(wbuf_ref[0], 0, 0)
        p1_r1.wait_recv()
        cp_l = start_wcopy(SLOT_L, 1, 0)
        cp_r.wait()
        dots(wbuf_ref[1], 1, 0)
        p1_l1.wait_recv()
        p2_l = remote_send(hbm_ref.at[SLOT_R, :, cols(1)],
                           opp_ref.at[:, cols(1)], P2_L, left)
        cp_r = start_wcopy(SLOT_R, 1, 1)
        cp_l.wait()
        dots(wbuf_ref[0], 0, 1)
        pc_l = send_piece(0, PIECE_FROM_R, PC_L, left)
        cp_r.wait()
        dots(wbuf_ref[1], 1, 1)
        pc_r = send_piece(1, PIECE_FROM_L, PC_R, right)

        p2_r.wait_recv()
        dots(opp_ref[:, cols(0)], 2, 0)
        p2_l.wait_recv()
        dots(opp_ref[:, cols(1)], 2, 1)
        pc_o = send_piece(2, PIECE_FROM_O, PC_O, opp)

        pc_l.wait_recv()
        out_ref[pl.ds(right * m_per, m_per), :] = (
            pieces_ref[PIECE_FROM_R].astype(jnp.float32))
        pc_r.wait_recv()
        out_ref[pl.ds(left * m_per, m_per), :] = (
            pieces_ref[PIECE_FROM_L].astype(jnp.float32))
        pc_o.wait_recv()
        out_ref[pl.ds(opp * m_per, m_per), :] = (
            pieces_ref[PIECE_FROM_O].astype(jnp.float32))
        for r in (p1_r0, p1_r1, p1_l0, p1_l1, p2_r, p2_l, pc_l, pc_r, pc_o):
            r.wait_send()

    out_shapes = [
        jax.ShapeDtypeStruct((N_DEV * m_per, n_per), jnp.float32),
        jax.ShapeDtypeStruct((2, k, n_per), bf16),
    ]
    out, _ = pl.pallas_call(
        body,
        out_shape=out_shapes,
        in_specs=[
            pl.BlockSpec(memory_space=pltpu.MemorySpace.VMEM),
            pl.BlockSpec(memory_space=pltpu.MemorySpace.VMEM),
        ],
        out_specs=[
            pl.BlockSpec(memory_space=pltpu.MemorySpace.VMEM),
            pl.BlockSpec(memory_space=pltpu.MemorySpace.HBM),
        ],
        scratch_shapes=[
            pltpu.VMEM((k, n_per), bf16),
            pltpu.VMEM((k, n_per), bf16),
            pltpu.VMEM((2, k, n_half), bf16),
            pltpu.VMEM((3, m_per, n_per), bf16),
            pltpu.VMEM((3, m_per, n_per), bf16),
            pltpu.SemaphoreType.DMA((9,)),
            pltpu.SemaphoreType.DMA((9,)),
            pltpu.SemaphoreType.DMA((2,)),
        ],
        compiler_params=pltpu.CompilerParams(
            collective_id=0,
            vmem_limit_bytes=63 * 1024 * 1024,
        ),
    )(x, w_mat)
    return out


# device time: 114545 ns/iter; 5.0407x vs baseline; 1.0424x over previous
import jax
import jax.numpy as jnp
from jax import lax
from jax.experimental import pallas as pl
from jax.experimental.pallas import tpu as pltpu

N_DEV = 4


def _gelu(y):
    c = 0.7978845608028654
    return 0.5 * y * (1.0 + jnp.tanh(c * (y + 0.044715 * y * y * y)))


def kernel(x, w_mat):
    m_per, k = x.shape
    _, n_per = w_mat.shape
    half = m_per // 2
    n_half = n_per // 2
    bf16 = jnp.bfloat16

    SLOT_L, SLOT_R = 0, 1
    PIECE_FROM_R, PIECE_FROM_L, PIECE_FROM_O = 0, 1, 2
    (P1_R0, P1_R1, P1_L0, P1_L1,
     P2_R, P2_L, PC_L, PC_R, PC_O) = range(9)

    def body(x_ref, w_ref, out_ref, hbm_ref,
             wsrc_ref, xbf_ref, xtmp_ref, opp_ref, wbuf_ref, stage_ref,
             pieces_ref, send_sems, recv_sems, copy_sems):
        my = lax.axis_index("i")
        left = (my - 1) % N_DEV
        right = (my + 1) % N_DEV
        opp = (my + 2) % N_DEV

        wsrc_ref[...] = w_ref[...].astype(bf16)

        barrier_sem = pltpu.get_barrier_semaphore()
        for nbr in (left, right, opp):
            pl.semaphore_signal(
                barrier_sem, inc=1,
                device_id=(nbr,), device_id_type=pl.DeviceIdType.MESH,
            )
        pl.semaphore_wait(barrier_sem, 3)

        def remote_send(src, dst, sem_idx, target):
            rdma = pltpu.make_async_remote_copy(
                src_ref=src,
                dst_ref=dst,
                send_sem=send_sems.at[sem_idx],
                recv_sem=recv_sems.at[sem_idx],
                device_id=(target,),
                device_id_type=pl.DeviceIdType.MESH,
            )
            rdma.start()
            return rdma

        def cols(hv):
            return pl.ds(hv * n_half, n_half)

        p1_r0 = remote_send(wsrc_ref.at[:, cols(0)],
                            hbm_ref.at[SLOT_L, :, cols(0)], P1_R0, right)
        p1_l0 = remote_send(wsrc_ref.at[:, cols(0)],
                            hbm_ref.at[SLOT_R, :, cols(0)], P1_L0, left)
        p1_r1 = remote_send(wsrc_ref.at[:, cols(1)],
                            hbm_ref.at[SLOT_L, :, cols(1)], P1_R1, right)
        p1_l1 = remote_send(wsrc_ref.at[:, cols(1)],
                            hbm_ref.at[SLOT_R, :, cols(1)], P1_L1, left)

        q = m_per // 4
        for t in range(4):
            cp = pltpu.make_async_copy(
                x_ref.at[pl.ds(t * q, q), :], xtmp_ref, copy_sems.at[2])
            cp.start()
            cp.wait()
            xbf_ref[pl.ds(t * q, q), :] = xtmp_ref[...].astype(bf16)

        for t in range(2):
            out_ref[pl.ds(my * m_per + t * half, half), :] = _gelu(
                jnp.dot(xbf_ref[pl.ds(t * half, half), :], wsrc_ref[...],
                        preferred_element_type=jnp.float32)
            )

        def dots(w_block, stage_idx, hv):
            for t in range(2):
                stage_ref[stage_idx, pl.ds(t * half, half), cols(hv)] = _gelu(
                    jnp.dot(xbf_ref[pl.ds(t * half, half), :], w_block,
                            preferred_element_type=jnp.float32)
                ).astype(bf16)

        def start_wcopy(slot, hv, wslot):
            cp = pltpu.make_async_copy(
                hbm_ref.at[slot, :, cols(hv)],
                wbuf_ref.at[wslot],
                copy_sems.at[wslot],
            )
            cp.start()
            return cp

        def send_piece(stage_idx, dst_slot, sem_idx, target):
            return remote_send(
                stage_ref.at[stage_idx],
                pieces_ref.at[dst_slot],
                sem_idx, target,
            )

        p1_r0.wait_recv()
        p2_r = remote_send(hbm_ref.at[SLOT_L, :, cols(0)],
                           opp_ref.at[:, cols(0)], P2_R, right)
        cp_l = start_wcopy(SLOT_L, 0, 0)
        p1_l0.wait_recv()
        cp_r = start_wcopy(SLOT_R, 0, 1)
        cp_l.wait()
        dots(wbuf_ref[0], 0, 0)
        p1_r1.wait_recv()
        cp_l = start_wcopy(SLOT_L, 1, 0)
        cp_r.wait()
        dots(wbuf_ref[1], 1, 0)
        p1_l1.wait_recv()
        p2_l = remote_send(hbm_ref.at[SLOT_R, :, cols(1)],
                           opp_ref.at[:, cols(1)], P2_L, left)
        cp_r = start_wcopy(SLOT_R, 1, 1)
        cp_l.wait()
        dots(wbuf_ref[0], 0, 1)
        pc_l = send_piece(0, PIECE_FROM_R, PC_L, left)
        cp_r.wait()
        dots(wbuf_ref[1], 1, 1)
        pc_r = send_piece(1, PIECE_FROM_L, PC_R, right)

        p2_r.wait_recv()
        dots(opp_ref[:, cols(0)], 2, 0)
        p2_l.wait_recv()
        dots(opp_ref[:, cols(1)], 2, 1)
        pc_o = send_piece(2, PIECE_FROM_O, PC_O, opp)

        pc_l.wait_recv()
        out_ref[pl.ds(right * m_per, m_per), :] = (
            pieces_ref[PIECE_FROM_R].astype(jnp.float32))
        pc_r.wait_recv()
        out_ref[pl.ds(left * m_per, m_per), :] = (
            pieces_ref[PIECE_FROM_L].astype(jnp.float32))
        pc_o.wait_recv()
        out_ref[pl.ds(opp * m_per, m_per), :] = (
            pieces_ref[PIECE_FROM_O].astype(jnp.float32))
        for r in (p1_r0, p1_r1, p1_l0, p1_l1, p2_r, p2_l, pc_l, pc_r, pc_o):
            r.wait_send()

    out_shapes = [
        jax.ShapeDtypeStruct((N_DEV * m_per, n_per), jnp.float32),
        jax.ShapeDtypeStruct((2, k, n_per), bf16),
    ]
    out, _ = pl.pallas_call(
        body,
        out_shape=out_shapes,
        in_specs=[
            pl.BlockSpec(memory_space=pltpu.MemorySpace.HBM),
            pl.BlockSpec(memory_space=pltpu.MemorySpace.VMEM),
        ],
        out_specs=[
            pl.BlockSpec(memory_space=pltpu.MemorySpace.VMEM),
            pl.BlockSpec(memory_space=pltpu.MemorySpace.HBM),
        ],
        scratch_shapes=[
            pltpu.VMEM((k, n_per), bf16),
            pltpu.VMEM((m_per, k), bf16),
            pltpu.VMEM((m_per // 4, k), jnp.float32),
            pltpu.VMEM((k, n_per), bf16),
            pltpu.VMEM((2, k, n_half), bf16),
            pltpu.VMEM((3, m_per, n_per), bf16),
            pltpu.VMEM((3, m_per, n_per), bf16),
            pltpu.SemaphoreType.DMA((9,)),
            pltpu.SemaphoreType.DMA((9,)),
            pltpu.SemaphoreType.DMA((3,)),
        ],
        compiler_params=pltpu.CompilerParams(
            collective_id=0,
            vmem_limit_bytes=63 * 1024 * 1024,
        ),
    )(x, w_mat)
    return out
